# Initial kernel scaffold; baseline (speedup 1.0000x reference)
import functools

import jax
import jax.numpy as jnp
from jax import lax
from jax.experimental import pallas as pl
from jax.experimental.pallas import tpu as pltpu
from jax.experimental.pallas import tpu_sc as plsc

_N = 10000
_E = 320000
_D = 128
_H = 128
_G = 32
_NC = 2
_NS = 16
_NW = _NC * _NS
_EPT = _E // _NW
_C = 100
_NCH = _EPT // _C
_RPT = _N // _NS


def _seg_sum_sc(x, src3, dst3, zeros):
  mesh = plsc.VectorSubcoreMesh(core_axis_name="c", subcore_axis_name="s")

  @functools.partial(
      pl.kernel,
      out_type=jax.ShapeDtypeStruct((_NC, _N, _D), jnp.float32),
      mesh=mesh,
      scratch_types=[
          pltpu.VMEM((_NCH, _C), jnp.int32),
          pltpu.VMEM((_NCH, _C), jnp.int32),
          pltpu.VMEM((_C, _D), jnp.float32),
          pltpu.VMEM_SHARED((_N, _D), jnp.float32),
          pltpu.SemaphoreType.DMA,
      ],
  )
  def seg_kernel(x_hbm, src_hbm, dst_hbm, z_hbm, out_hbm, src_v, dst_v,
                 rows_v, acc, sem):
    cid = lax.axis_index("c")
    sid = lax.axis_index("s")
    wid = cid * _NS + sid
    pltpu.sync_copy(z_hbm, acc.at[pl.ds(sid * _RPT, _RPT)])
    pltpu.sync_copy(src_hbm.at[wid], src_v)
    pltpu.sync_copy(dst_hbm.at[wid], dst_v)
    plsc.subcore_barrier()

    def body(j, carry):
      pltpu.async_copy(x_hbm.at[src_v.at[j]], rows_v, sem).wait()
      pltpu.sync_copy(rows_v, acc.at[dst_v.at[j]], add=True)
      return carry

    lax.fori_loop(0, _NCH, body, 0)

    plsc.subcore_barrier()
    pltpu.sync_copy(acc.at[pl.ds(sid * _RPT, _RPT)],
                    out_hbm.at[cid, pl.ds(sid * _RPT, _RPT)])

  return seg_kernel(x, src3, dst3, zeros)


def _pre_tc(nf, Wpre, bpre):
  def body(x_ref, w_ref, b_ref, o_ref):
    o_ref[...] = (
        jnp.dot(x_ref[...], w_ref[...], preferred_element_type=jnp.float32)
        + b_ref[...])

  return pl.pallas_call(
      body, out_shape=jax.ShapeDtypeStruct((_N, _H), jnp.float32),
  )(nf, Wpre, bpre)


def _dense_bn_tc(x, agg, W1, b1, W2, b2, gamma, beta):
  def body(x_ref, a_ref, w1_ref, b1_ref, w2_ref, b2_ref, g_ref, be_ref,
           o_ref):
    h = x_ref[...] + a_ref[0] + a_ref[1]
    h = jnp.maximum(
        jnp.dot(h, w1_ref[...], preferred_element_type=jnp.float32)
        + b1_ref[...], 0.0)
    h = (jnp.dot(h, w2_ref[...], preferred_element_type=jnp.float32)
         + b2_ref[...])
    mean = jnp.mean(h, axis=0, keepdims=True)
    var = jnp.mean(h * h, axis=0, keepdims=True) - mean * mean
    o_ref[...] = (g_ref[...] * (h - mean) * lax.rsqrt(var + 1e-5)
                  + be_ref[...])

  return pl.pallas_call(
      body, out_shape=jax.ShapeDtypeStruct((_N, _H), jnp.float32),
  )(x, agg, W1, b1, W2, b2, gamma, beta)


def _final_tc(x, agg, W1, b1, W2, b2, batch2d, Wp1, bp1, Wp2, bp2):
  def body(x_ref, a_ref, w1_ref, b1_ref, w2_ref, b2_ref, bt_ref, wp1_ref,
           bp1_ref, wp2_ref, bp2_ref, o_ref):
    h = x_ref[...] + a_ref[0] + a_ref[1]
    h = jnp.maximum(
        jnp.dot(h, w1_ref[...], preferred_element_type=jnp.float32)
        + b1_ref[...], 0.0)
    h = (jnp.dot(h, w2_ref[...], preferred_element_type=jnp.float32)
         + b2_ref[...])
    onehot = (bt_ref[...] == lax.broadcasted_iota(jnp.int32, (1, _G), 1)
              ).astype(jnp.float32)
    sums = lax.dot_general(onehot, h, (((0,), (0,)), ((), ())),
                           preferred_element_type=jnp.float32)
    counts = jnp.sum(onehot, axis=0).reshape(_G, 1)
    pooled = sums / jnp.maximum(counts, 1.0)
    hh = jnp.maximum(
        jnp.dot(pooled, wp1_ref[...], preferred_element_type=jnp.float32)
        + bp1_ref[...], 0.0)
    o_ref[...] = (
        jnp.dot(hh, wp2_ref[...], preferred_element_type=jnp.float32)
        + bp2_ref[...])

  return pl.pallas_call(
      body, out_shape=jax.ShapeDtypeStruct((_G, 1), jnp.float32),
  )(x, agg, W1, b1, W2, b2, batch2d, Wp1, bp1, Wp2, bp2)


def kernel(node_feature, edge_index, batch, Wpre, bpre, W1s, b1s, W2s, b2s,
           gammas, betas, Wp1, bp1, Wp2, bp2):
  src3 = edge_index[0].reshape(_NW, _NCH, _C)
  dst3 = edge_index[1].reshape(_NW, _NCH, _C)
  zeros = jnp.zeros((_RPT, _D), jnp.float32)
  batch2d = batch.reshape(_N, 1)

  x = _pre_tc(node_feature, Wpre, bpre.reshape(1, _H))
  for i in range(2):
    agg = _seg_sum_sc(x, src3, dst3, zeros)
    x = _dense_bn_tc(x, agg, W1s[i], b1s[i].reshape(1, _H), W2s[i],
                     b2s[i].reshape(1, _H), gammas[i].reshape(1, _H),
                     betas[i].reshape(1, _H))
  agg = _seg_sum_sc(x, src3, dst3, zeros)
  return _final_tc(x, agg, W1s[2], b1s[2].reshape(1, _H), W2s[2],
                   b2s[2].reshape(1, _H), batch2d, Wp1, bp1.reshape(1, _H),
                   Wp2, bp2.reshape(1, 1))


# speedup vs baseline: 7.1482x; 7.1482x over previous
import functools

import jax
import jax.numpy as jnp
from jax import lax
from jax.experimental import pallas as pl
from jax.experimental.pallas import tpu as pltpu
from jax.experimental.pallas import tpu_sc as plsc

_N = 10000
_E = 320000
_D = 128
_H = 128
_G = 32
_NC = 2
_NS = 16
_NW = _NC * _NS
_EPT = _E // _NW
_C = 100
_NCH = _EPT // _C
_NP = 10240
_RPT = _NP // _NS


def _seg_sum_sc(x, src3, dst3, zeros):
  mesh = plsc.VectorSubcoreMesh(core_axis_name="c", subcore_axis_name="s")

  @functools.partial(
      pl.kernel,
      out_type=jax.ShapeDtypeStruct((_NC, _NP, _D), jnp.float32),
      mesh=mesh,
      scratch_types=[
          pltpu.VMEM((_NCH, _C), jnp.int32),
          pltpu.VMEM((_NCH, _C), jnp.int32),
          pltpu.VMEM((_C, _D), jnp.float32),
          pltpu.VMEM_SHARED((_NP, _D), jnp.float32),
          pltpu.SemaphoreType.DMA,
      ],
  )
  def seg_kernel(x_hbm, src_hbm, dst_hbm, z_hbm, out_hbm, src_v, dst_v,
                 rows_v, acc, sem):
    cid = lax.axis_index("c")
    sid = lax.axis_index("s")
    wid = cid * _NS + sid
    pltpu.sync_copy(z_hbm, acc.at[pl.ds(sid * _RPT, _RPT)])
    pltpu.sync_copy(src_hbm.at[wid], src_v)
    pltpu.sync_copy(dst_hbm.at[wid], dst_v)
    plsc.subcore_barrier()

    def body(j, carry):
      pltpu.async_copy(x_hbm.at[src_v.at[j]], rows_v, sem).wait()
      pltpu.sync_copy(rows_v, acc.at[dst_v.at[j]], add=True)
      return carry

    lax.fori_loop(0, _NCH, body, 0)

    plsc.subcore_barrier()
    pltpu.sync_copy(acc.at[pl.ds(sid * _RPT, _RPT)],
                    out_hbm.at[cid, pl.ds(sid * _RPT, _RPT)])

  return seg_kernel(x, src3, dst3, zeros)


def _pre_tc(nf, Wpre, bpre):
  def body(x_ref, w_ref, b_ref, o_ref):
    o_ref[...] = (
        jnp.dot(x_ref[...], w_ref[...], preferred_element_type=jnp.float32)
        + b_ref[...])

  return pl.pallas_call(
      body, out_shape=jax.ShapeDtypeStruct((_N, _H), jnp.float32),
  )(nf, Wpre, bpre)


def _dense_bn_tc(x, agg, W1, b1, W2, b2, gamma, beta):
  def body(x_ref, a_ref, w1_ref, b1_ref, w2_ref, b2_ref, g_ref, be_ref,
           o_ref):
    h = x_ref[...] + a_ref[0, :_N, :] + a_ref[1, :_N, :]
    h = jnp.maximum(
        jnp.dot(h, w1_ref[...], preferred_element_type=jnp.float32)
        + b1_ref[...], 0.0)
    h = (jnp.dot(h, w2_ref[...], preferred_element_type=jnp.float32)
         + b2_ref[...])
    mean = jnp.mean(h, axis=0, keepdims=True)
    var = jnp.mean(h * h, axis=0, keepdims=True) - mean * mean
    o_ref[...] = (g_ref[...] * (h - mean) * lax.rsqrt(var + 1e-5)
                  + be_ref[...])

  return pl.pallas_call(
      body, out_shape=jax.ShapeDtypeStruct((_N, _H), jnp.float32),
  )(x, agg, W1, b1, W2, b2, gamma, beta)


def _final_tc(x, agg, W1, b1, W2, b2, batch2d, Wp1, bp1, Wp2, bp2):
  def body(x_ref, a_ref, w1_ref, b1_ref, w2_ref, b2_ref, bt_ref, wp1_ref,
           bp1_ref, wp2_ref, bp2_ref, o_ref):
    h = x_ref[...] + a_ref[0, :_N, :] + a_ref[1, :_N, :]
    h = jnp.maximum(
        jnp.dot(h, w1_ref[...], preferred_element_type=jnp.float32)
        + b1_ref[...], 0.0)
    h = (jnp.dot(h, w2_ref[...], preferred_element_type=jnp.float32)
         + b2_ref[...])
    onehot = (bt_ref[...] == lax.broadcasted_iota(jnp.int32, (1, _G), 1)
              ).astype(jnp.float32)
    sums = lax.dot_general(onehot, h, (((0,), (0,)), ((), ())),
                           preferred_element_type=jnp.float32)
    counts = jnp.sum(onehot, axis=0).reshape(_G, 1)
    pooled = sums / jnp.maximum(counts, 1.0)
    hh = jnp.maximum(
        jnp.dot(pooled, wp1_ref[...], preferred_element_type=jnp.float32)
        + bp1_ref[...], 0.0)
    o_ref[...] = (
        jnp.dot(hh, wp2_ref[...], preferred_element_type=jnp.float32)
        + bp2_ref[...])

  return pl.pallas_call(
      body, out_shape=jax.ShapeDtypeStruct((_G, 1), jnp.float32),
  )(x, agg, W1, b1, W2, b2, batch2d, Wp1, bp1, Wp2, bp2)


def kernel(node_feature, edge_index, batch, Wpre, bpre, W1s, b1s, W2s, b2s,
           gammas, betas, Wp1, bp1, Wp2, bp2):
  src3 = edge_index[0].reshape(_NW, _NCH, _C)
  dst3 = edge_index[1].reshape(_NW, _NCH, _C)
  zeros = jnp.zeros((_RPT, _D), jnp.float32)
  batch2d = batch.reshape(_N, 1)

  x = _pre_tc(node_feature, Wpre, bpre.reshape(1, _H))
  for i in range(2):
    agg = _seg_sum_sc(x, src3, dst3, zeros)
    x = _dense_bn_tc(x, agg, W1s[i], b1s[i].reshape(1, _H), W2s[i],
                     b2s[i].reshape(1, _H), gammas[i].reshape(1, _H),
                     betas[i].reshape(1, _H))
  agg = _seg_sum_sc(x, src3, dst3, zeros)
  return _final_tc(x, agg, W1s[2], b1s[2].reshape(1, _H), W2s[2],
                   b2s[2].reshape(1, _H), batch2d, Wp1, bp1.reshape(1, _H),
                   Wp2, bp2.reshape(1, 1))
